# Initial kernel scaffold; baseline (speedup 1.0000x reference)
import jax
import jax.numpy as jnp
from jax.experimental import pallas as pl


def kernel(logits, A_hat_indices, A_hat_values):
    raise NotImplementedError("write your pallas kernel here")


# speedup vs baseline: 1.5275x; 1.5275x over previous
import functools

import jax
import jax.numpy as jnp
from jax import lax
from jax.experimental import pallas as pl
from jax.experimental.pallas import tpu as pltpu
from jax.experimental.pallas import tpu_sc as plsc

ALPHA = 0.15
N_PROP = 5
LANES = 16
NSUB = 16
CHUNK = 12800
STRIPE = CHUNK // NSUB
CSLICE = 80
EB = 2000
GB = 128
CB = 2048
NF = CB // GB


@functools.cache
def _make_step(n, c, e):
    assert c == 64, "kernel specialized for 64 channels"
    ept = e // NSUB
    assert e % NSUB == 0 and ept % EB == 0
    nb = ept // EB
    n_slices = STRIPE // CSLICE
    npass = -(-n // (2 * CHUNK))
    mesh = plsc.VectorSubcoreMesh(core_axis_name="c", subcore_axis_name="s")

    def body(x_hbm, a_hbm, rows_hbm, cols_hbm, vals_hbm, out_hbm,
             acc, rows_v, cols_v, vals_v, crows, ccols, cvals,
             grows, obuf, abuf, ldsem, gsem):
        cid = lax.axis_index("c")
        sid = lax.axis_index("s")

        def zi(i, _):
            rr = i // (GB // LANES)
            cc = (i % (GB // LANES)) * LANES
            z = jnp.zeros((LANES,), jnp.int32)
            crows[rr, pl.ds(cc, LANES)] = z
            ccols[rr, pl.ds(cc, LANES)] = z
            cvals[rr, pl.ds(cc, LANES)] = jnp.zeros((LANES,), jnp.float32)
            return 0
        lax.fori_loop(0, CB // LANES, zi, 0)

        for jj in range(npass):
            base = (cid * npass + jj) * CHUNK

            def zo(i, _):
                obuf[i // 4, pl.ds((i % 4) * LANES, LANES)] = (
                    jnp.zeros((LANES,), jnp.float32))
                return 0
            lax.fori_loop(0, CSLICE * 4, zo, 0)
            for k in range(n_slices):
                pltpu.sync_copy(
                    obuf, acc.at[pl.ds(sid * STRIPE + k * CSLICE, CSLICE)])
            plsc.subcore_barrier()

            def batch_body(b, _):
                ebase = sid * ept + b * EB
                cp1 = pltpu.async_copy(rows_hbm.at[pl.ds(ebase, EB)], rows_v, ldsem)
                cp2 = pltpu.async_copy(cols_hbm.at[pl.ds(ebase, EB)], cols_v, ldsem)
                cp3 = pltpu.async_copy(vals_hbm.at[pl.ds(ebase, EB)], vals_v, ldsem)
                cp1.wait(); cp2.wait(); cp3.wait()

                def grp(g, cur):
                    r = rows_v[pl.ds(g * LANES, LANES)]
                    base_v = jnp.full((LANES,), base, jnp.int32)
                    hi_v = jnp.full((LANES,), base + CHUNK, jnp.int32)
                    m = (r >= base_v) & (r < hi_v)
                    mi = m.astype(jnp.int32)
                    pos = jnp.full((LANES,), cur, jnp.int32) + plsc.cumsum(mi) - mi
                    pf = pos >> jnp.full((LANES,), 7, jnp.int32)
                    pe = pos & jnp.full((LANES,), GB - 1, jnp.int32)
                    plsc.store_scatter(crows, [pf, pe], r - base_v, mask=m)
                    plsc.store_scatter(
                        ccols, [pf, pe], cols_v[pl.ds(g * LANES, LANES)], mask=m)
                    plsc.store_scatter(
                        cvals, [pf, pe], vals_v[pl.ds(g * LANES, LANES)], mask=m)
                    return cur + jnp.sum(mi)
                cur = lax.fori_loop(0, EB // LANES, grp, jnp.int32(0))

                nf = (cur + (GB - 1)) // GB
                def flush(f, _):
                    pltpu.async_copy(x_hbm.at[ccols.at[f]], grows, gsem).wait()

                    def scl(i, _):
                        eidx = f * GB + i
                        vv = plsc.load_gather(
                            cvals,
                            [jnp.full((LANES,), f, jnp.int32),
                             jnp.full((LANES,), i, jnp.int32)])
                        vv = vv * jnp.full(
                            (LANES,), (eidx < cur).astype(jnp.float32))
                        for cg in range(4):
                            grows[i, pl.ds(cg * LANES, LANES)] = (
                                grows[i, pl.ds(cg * LANES, LANES)] * vv)
                        return 0
                    lax.fori_loop(0, GB, scl, 0)
                    pltpu.sync_copy(grows, acc.at[crows.at[f]], add=True)
                    return 0
                lax.fori_loop(0, nf, flush, 0)
                return 0
            lax.fori_loop(0, nb, batch_body, 0)
            plsc.subcore_barrier()

            for k in range(n_slices):
                gbase = base + sid * STRIPE + k * CSLICE

                @pl.when(gbase < n)
                def _():
                    pltpu.sync_copy(
                        acc.at[pl.ds(sid * STRIPE + k * CSLICE, CSLICE)], obuf)
                    pltpu.async_copy(
                        a_hbm.at[pl.ds(gbase, CSLICE)], abuf, ldsem).wait()

                    def cmb(i, _):
                        rr = i // 4
                        cg = (i % 4) * LANES
                        o = obuf[rr, pl.ds(cg, LANES)]
                        av = abuf[rr, pl.ds(cg, LANES)]
                        oa = jnp.full((LANES,), 1.0 - ALPHA, jnp.float32)
                        aa = jnp.full((LANES,), ALPHA, jnp.float32)
                        obuf[rr, pl.ds(cg, LANES)] = oa * o + aa * av
                        return 0
                    lax.fori_loop(0, CSLICE * 4, cmb, 0)
                    pltpu.sync_copy(obuf, out_hbm.at[pl.ds(gbase, CSLICE)])
            plsc.subcore_barrier()

    return pl.kernel(
        body,
        out_type=jax.ShapeDtypeStruct((n, c), jnp.float32),
        mesh=mesh,
        compiler_params=pltpu.CompilerParams(
            needs_layout_passes=False, use_tc_tiling_on_sc=False),
        scratch_types=[
            pltpu.VMEM_SHARED((CHUNK, 64), jnp.float32),
            pltpu.VMEM((EB,), jnp.int32),
            pltpu.VMEM((EB,), jnp.int32),
            pltpu.VMEM((EB,), jnp.float32),
            pltpu.VMEM((NF, GB), jnp.int32),
            pltpu.VMEM((NF, GB), jnp.int32),
            pltpu.VMEM((NF, GB), jnp.float32),
            pltpu.VMEM((GB, 64), jnp.float32),
            pltpu.VMEM((CSLICE, 64), jnp.float32),
            pltpu.VMEM((CSLICE, 64), jnp.float32),
            pltpu.SemaphoreType.DMA,
            pltpu.SemaphoreType.DMA,
        ],
    )


def kernel(logits, A_hat_indices, A_hat_values):
    n, c = logits.shape
    e = A_hat_values.shape[0]
    step = _make_step(n, c, e)
    rows = A_hat_indices[0]
    cols = A_hat_indices[1]
    x = logits
    for _ in range(N_PROP):
        x = step(x, logits, rows, cols, A_hat_values)
    return x
